# baseline (speedup 1.0000x reference)
import functools

import jax
import jax.numpy as jnp
from jax.experimental import pallas as pl
from jax.experimental.pallas import tpu as pltpu

_BR = 320
_NBUF = 3


def _block_dma(j, e_ref, a_ref, bufs, sems, g, m, do_wait):
    slot = jax.lax.rem(j, _NBUF)
    tail = m - (g - 1) * _BR
    is_e = j < g
    jl = jnp.where(is_e, j, j - g)
    not_tail = jl < g - 1

    def run(copy):
        copy.wait() if do_wait else copy.start()

    for k in range(_NBUF):
        on = slot == k
        buf = bufs[k]
        sem = sems.at[k]

        @pl.when(on & is_e & not_tail)
        def _(buf=buf, sem=sem):
            run(pltpu.make_async_copy(
                e_ref.at[pl.ds(jl * _BR, _BR), :], buf, sem))

        @pl.when(on & is_e & ~not_tail)
        def _(buf=buf, sem=sem):
            run(pltpu.make_async_copy(
                e_ref.at[pl.ds((g - 1) * _BR, tail), :],
                buf.at[0:tail, :], sem))

        @pl.when(on & ~is_e & not_tail)
        def _(buf=buf, sem=sem):
            run(pltpu.make_async_copy(
                a_ref.at[pl.ds(jl * _BR, _BR), :], buf, sem))

        @pl.when(on & ~is_e & ~not_tail)
        def _(buf=buf, sem=sem):
            run(pltpu.make_async_copy(
                a_ref.at[pl.ds((g - 1) * _BR, tail), :],
                buf.at[0:tail, :], sem))


def _fused_kernel(
    e_ref, a_ref, x_ref, y_ref, wfc_ref, bfc_ref, w1_ref, b1_ref, w2_ref,
    b2_ref, o_ref, xe_ref, *scratch, g, m
):
    bufs = scratch[:_NBUF]
    sems, s1_scr, s2_scr = scratch[_NBUF:]
    i = pl.program_id(0)

    @pl.when(i == 0)
    def _startup():
        for j in range(_NBUF - 1):
            pltpu.make_async_copy(
                e_ref.at[pl.ds(j * _BR, _BR), :], bufs[j], sems.at[j]
            ).start()
        y_new = (
            jnp.dot(y_ref[:], wfc_ref[:], preferred_element_type=jnp.float32)
            + bfc_ref[:]
        )
        n_nodes = x_ref.shape[0]
        s1_scr[0:n_nodes, :] = jnp.dot(
            x_ref[:], w1_ref[:], preferred_element_type=jnp.float32
        )
        s1_scr[n_nodes:, :] = jnp.dot(
            y_new, w1_ref[:], preferred_element_type=jnp.float32
        )

    nxt = i + _NBUF - 1

    @pl.when(nxt < 2 * g)
    def _prefetch():
        _block_dma(nxt, e_ref, a_ref, bufs, sems, g, m, do_wait=False)

    _block_dma(i, e_ref, a_ref, bufs, sems, g, m, do_wait=True)

    slot = jax.lax.rem(i, _NBUF)

    def _compute(blk_ref):
        @pl.when(i < g)
        def _phase1():
            acc = jnp.dot(
                blk_ref[:], s1_scr[:], preferred_element_type=jnp.float32
            )
            xe = jnp.maximum(acc + b1_ref[:], 0.0)
            xe_ref[:] = xe
            s2_scr[pl.ds(i * _BR, _BR), :] = jnp.dot(
                xe, w2_ref[:], preferred_element_type=jnp.float32
            )

        @pl.when(i >= g)
        def _phase2():
            acc = jnp.dot(
                blk_ref[:], s2_scr[0:m, :], preferred_element_type=jnp.float32
            )
            o_ref[:] = jax.nn.sigmoid(acc + b2_ref[:])

    for k in range(_NBUF):
        @pl.when(slot == k)
        def _(k=k):
            _compute(bufs[k])


def kernel(Y_embedding, X, E_tilde, A_tilde, W_fc, b_fc, W1, b1, W2, b2):
    m = E_tilde.shape[0]
    n = X.shape[0]
    nfeat = X.shape[1]
    nhid = W1.shape[1]
    nclass = W2.shape[1]
    nhigh = Y_embedding.shape[1]
    l = Y_embedding.shape[0]
    f32 = jnp.float32

    bfc2 = b_fc.reshape(1, nfeat)
    b1_2 = b1.reshape(1, nhid)
    b2_2 = b2.reshape(1, nclass)

    g = pl.cdiv(m, _BR)

    const = lambda i: (0, 0)
    hbm = pl.BlockSpec(memory_space=pltpu.MemorySpace.HBM)
    body = functools.partial(_fused_kernel, g=g, m=m)

    output, x_embedding = pl.pallas_call(
        body,
        grid=(2 * g,),
        in_specs=[
            hbm,
            hbm,
            pl.BlockSpec((n, nfeat), const),
            pl.BlockSpec((l, nhigh), const),
            pl.BlockSpec((nhigh, nfeat), const),
            pl.BlockSpec((1, nfeat), const),
            pl.BlockSpec((nfeat, nhid), const),
            pl.BlockSpec((1, nhid), const),
            pl.BlockSpec((nhid, nclass), const),
            pl.BlockSpec((1, nclass), const),
        ],
        out_specs=[
            pl.BlockSpec((_BR, nclass), lambda i: (jnp.maximum(i - g, 0), 0)),
            pl.BlockSpec((_BR, nhid), lambda i: (jnp.minimum(i, g - 1), 0)),
        ],
        out_shape=[
            jax.ShapeDtypeStruct((m, nclass), f32),
            jax.ShapeDtypeStruct((m, nhid), f32),
        ],
        scratch_shapes=(
            [pltpu.VMEM((_BR, m), f32) for _ in range(_NBUF)]
            + [
                pltpu.SemaphoreType.DMA((_NBUF,)),
                pltpu.VMEM((m, nhid), f32),
                pltpu.VMEM((g * _BR, nclass), f32),
            ]
        ),
        compiler_params=pltpu.CompilerParams(
            dimension_semantics=("arbitrary",)
        ),
    )(E_tilde, A_tilde, X, Y_embedding, W_fc, bfc2, W1, b1_2, W2, b2_2)

    return (output, x_embedding)
